# baseline (speedup 1.0000x reference)
import jax
import jax.numpy as jnp
from jax import lax
from jax.experimental import pallas as pl
from jax.experimental.pallas import tpu as pltpu
from jax.experimental.pallas import tpu_sc as plsc

B = 16384
D = 32
NC = 2
NS = 16
L = 16
NW = NC * NS
BPW = B // NW
CHUNK = 128
NCHUNK = BPW // CHUNK
NGROUP = BPW // L
Y_LO, Y_HI = 1.0, 5.0


def _mf_kernel(users_hbm, items_hbm, ue_hbm, ie_hbm, uet_hbm, ub_hbm, ib_hbm,
               out_hbm, idx_u, idx_i, ue_v, ie_v, ub_v, ib_v, out_v, sem):
    wid = lax.axis_index("s") * NC + lax.axis_index("c")

    pltpu.sync_copy(users_hbm.at[wid], idx_u)
    pltpu.sync_copy(items_hbm.at[wid], idx_i)

    copies = []
    for j in range(NCHUNK):
        rows = pl.ds(j * CHUNK, CHUNK)
        copies.append(pltpu.make_async_copy(
            ue_hbm.at[idx_u.at[j]], ue_v.at[rows], sem))
        copies.append(pltpu.make_async_copy(
            ie_hbm.at[idx_i.at[j]], ie_v.at[rows], sem))
        copies.append(pltpu.make_async_copy(
            ub_hbm.at[idx_u.at[j]], ub_v.at[rows], sem))
        copies.append(pltpu.make_async_copy(
            ib_hbm.at[idx_i.at[j]], ib_v.at[rows], sem))
    for c in copies:
        c.start()
    for c in copies:
        c.wait()

    def acc_body(g, carry):
        base = pl.multiple_of(g * L, L)
        out_v[pl.ds(base, L)] = ub_v[pl.ds(base, L)] + ib_v[pl.ds(base, L)]
        for k in range(L):
            r = base + k
            part = (ue_v[r, pl.ds(0, L)] * ie_v[r, pl.ds(0, L)]
                    + ue_v[r, pl.ds(L, L)] * ie_v[r, pl.ds(L, L)])
            idx = jnp.full((L,), r, jnp.int32)
            plsc.addupdate_scatter(out_v, [idx], part)
        return carry

    lax.fori_loop(0, NGROUP, acc_body, 0)

    def act_body(g, carry):
        base = pl.multiple_of(g * L, L)
        acc = out_v[pl.ds(base, L)]
        y = 1.0 / (1.0 + jnp.exp(-acc))
        out_v[pl.ds(base, L)] = y * (Y_HI - Y_LO) + Y_LO
        return carry

    lax.fori_loop(0, NGROUP, act_body, 0)

    pltpu.sync_copy(out_v, out_hbm.at[pl.ds(wid * BPW, BPW)])


@jax.jit
def kernel(users, items, user_emb, item_emb, user_bias, item_bias):
    users = users.astype(jnp.int32).reshape(NW, NCHUNK, CHUNK)
    items = items.astype(jnp.int32).reshape(NW, NCHUNK, CHUNK)
    uet = user_emb.T
    ub = user_bias.reshape(-1)
    ib = item_bias.reshape(-1)

    mesh = plsc.VectorSubcoreMesh(core_axis_name="c", subcore_axis_name="s")
    run = pl.kernel(
        _mf_kernel,
        out_type=jax.ShapeDtypeStruct((B,), jnp.float32),
        mesh=mesh,
        compiler_params=pltpu.CompilerParams(
            needs_layout_passes=False, use_tc_tiling_on_sc=False),
        scratch_types=[
            pltpu.VMEM((NCHUNK, CHUNK), jnp.int32),
            pltpu.VMEM((NCHUNK, CHUNK), jnp.int32),
            pltpu.VMEM((BPW, D), jnp.float32),
            pltpu.VMEM((BPW, D), jnp.float32),
            pltpu.VMEM((BPW,), jnp.float32),
            pltpu.VMEM((BPW,), jnp.float32),
            pltpu.VMEM((BPW,), jnp.float32),
            pltpu.SemaphoreType.DMA,
        ],
    )
    return run(users, items, user_emb, item_emb, uet, ub, ib)


# speedup vs baseline: 3.8168x; 3.8168x over previous
import jax
import jax.numpy as jnp
from jax import lax
from jax.experimental import pallas as pl
from jax.experimental.pallas import tpu as pltpu
from jax.experimental.pallas import tpu_sc as plsc

B = 16384
D = 32
NC = 2
NS = 16
L = 16
NW = NC * NS
BPW = B // NW
CHUNK = 128
NCHUNK = BPW // CHUNK
NGROUP = BPW // L
Y_LO, Y_HI = 1.0, 5.0


def _mf_kernel(users_hbm, items_hbm, ue_hbm, ie_hbm, ub_hbm, ib_hbm,
               out_hbm, idx_u, idx_i, ue_v, ie_v, ub_v, ib_v, out_v, sem):
    wid = lax.axis_index("s") * NC + lax.axis_index("c")

    pltpu.sync_copy(users_hbm.at[wid], idx_u)
    pltpu.sync_copy(items_hbm.at[wid], idx_i)

    copies = []
    for j in range(NCHUNK):
        rows = pl.ds(j * CHUNK, CHUNK)
        copies.append(pltpu.make_async_copy(
            ue_hbm.at[idx_u.at[j]], ue_v.at[rows], sem))
        copies.append(pltpu.make_async_copy(
            ie_hbm.at[idx_i.at[j]], ie_v.at[rows], sem))
        copies.append(pltpu.make_async_copy(
            ub_hbm.at[idx_u.at[j]], ub_v.at[rows], sem))
        copies.append(pltpu.make_async_copy(
            ib_hbm.at[idx_i.at[j]], ib_v.at[rows], sem))
    for c in copies:
        c.start()
    for c in copies:
        c.wait()

    def acc_body(g, carry):
        base = pl.multiple_of(g * L, L)
        out_v[pl.ds(base, L)] = ub_v[pl.ds(base, L)] + ib_v[pl.ds(base, L)]
        for k in range(L):
            r = base + k
            part = (ue_v[r, pl.ds(0, L)] * ie_v[r, pl.ds(0, L)]
                    + ue_v[r, pl.ds(L, L)] * ie_v[r, pl.ds(L, L)])
            idx = jnp.full((L,), r, jnp.int32)
            plsc.addupdate_scatter(out_v, [idx], part)
        return carry

    lax.fori_loop(0, NGROUP, acc_body, 0)

    def act_body(g, carry):
        base = pl.multiple_of(g * L, L)
        acc = out_v[pl.ds(base, L)]
        y = 1.0 / (1.0 + jnp.exp(-acc))
        out_v[pl.ds(base, L)] = y * (Y_HI - Y_LO) + Y_LO
        return carry

    lax.fori_loop(0, NGROUP, act_body, 0)

    pltpu.sync_copy(out_v, out_hbm.at[pl.ds(wid * BPW, BPW)])


@jax.jit
def kernel(users, items, user_emb, item_emb, user_bias, item_bias):
    users = users.astype(jnp.int32).reshape(NW, NCHUNK, CHUNK)
    items = items.astype(jnp.int32).reshape(NW, NCHUNK, CHUNK)
    ub = user_bias.reshape(-1)
    ib = item_bias.reshape(-1)

    mesh = plsc.VectorSubcoreMesh(core_axis_name="c", subcore_axis_name="s")
    run = pl.kernel(
        _mf_kernel,
        out_type=jax.ShapeDtypeStruct((B,), jnp.float32),
        mesh=mesh,
        compiler_params=pltpu.CompilerParams(
            needs_layout_passes=False, use_tc_tiling_on_sc=False),
        scratch_types=[
            pltpu.VMEM((NCHUNK, CHUNK), jnp.int32),
            pltpu.VMEM((NCHUNK, CHUNK), jnp.int32),
            pltpu.VMEM((BPW, D), jnp.float32),
            pltpu.VMEM((BPW, D), jnp.float32),
            pltpu.VMEM((BPW,), jnp.float32),
            pltpu.VMEM((BPW,), jnp.float32),
            pltpu.VMEM((BPW,), jnp.float32),
            pltpu.SemaphoreType.DMA,
        ],
    )
    return run(users, items, user_emb, item_emb, ub, ib)


# speedup vs baseline: 14.5997x; 3.8251x over previous
import jax
import jax.numpy as jnp
from jax import lax
from jax.experimental import pallas as pl
from jax.experimental.pallas import tpu as pltpu
from jax.experimental.pallas import tpu_sc as plsc

B = 16384
D = 32
NC = 2
NS = 16
L = 16
NW = NC * NS
BPW = B // NW
CHUNK = 128
NCHUNK = BPW // CHUNK
NGROUP = BPW // L
NSLOT = 8
NROUND = BPW // NSLOT
Y_LO, Y_HI = 1.0, 5.0


def _dot_kernel(users_hbm, items_hbm, uet_hbm, iet_hbm, dot_hbm,
                us_v, is_v, acc_v, *slots_and_sems):
    slots_u = slots_and_sems[0:NSLOT]
    slots_i = slots_and_sems[NSLOT:2 * NSLOT]
    sems_u = slots_and_sems[2 * NSLOT:3 * NSLOT]
    sems_i = slots_and_sems[3 * NSLOT:4 * NSLOT]

    wid = lax.axis_index("s") * NC + lax.axis_index("c")
    base_b = wid * BPW

    pltpu.sync_copy(users_hbm.at[pl.ds(base_b, BPW)], us_v)
    pltpu.sync_copy(items_hbm.at[pl.ds(base_b, BPW)], is_v)

    def zero_body(g, carry):
        acc_v[pl.ds(pl.multiple_of(g * L, L), L)] = jnp.zeros((L,), jnp.float32)
        return carry

    lax.fori_loop(0, NGROUP, zero_body, 0)

    iota = lax.iota(jnp.int32, L)

    def read_idx(ref, b):
        grp = pl.multiple_of((b // L) * L, L)
        v = ref[pl.ds(grp, L)]
        mask = iota == jnp.full((L,), b % L, jnp.int32)
        return jnp.sum(jnp.where(mask, v, 0))

    def issue(b, k):
        cu = pl.multiple_of((read_idx(us_v, b) >> 7) * 128, 128)
        ci = pl.multiple_of((read_idx(is_v, b) >> 7) * 128, 128)
        pltpu.make_async_copy(
            uet_hbm.at[:, pl.ds(cu, CHUNK)], slots_u[k], sems_u[k]).start()
        pltpu.make_async_copy(
            iet_hbm.at[:, pl.ds(ci, CHUNK)], slots_i[k], sems_i[k]).start()

    def consume(b, k):
        pltpu.make_async_copy(
            uet_hbm.at[:, pl.ds(0, CHUNK)], slots_u[k], sems_u[k]).wait()
        pltpu.make_async_copy(
            iet_hbm.at[:, pl.ds(0, CHUNK)], slots_i[k], sems_i[k]).wait()
        lu = jnp.full((L,), read_idx(us_v, b) & 127, jnp.int32)
        li = jnp.full((L,), read_idx(is_v, b) & 127, jnp.int32)
        gu_lo = plsc.load_gather(slots_u[k], [iota, lu])
        gu_hi = plsc.load_gather(slots_u[k], [iota + L, lu])
        gi_lo = plsc.load_gather(slots_i[k], [iota, li])
        gi_hi = plsc.load_gather(slots_i[k], [iota + L, li])
        part = gu_lo * gi_lo + gu_hi * gi_hi
        plsc.addupdate_scatter(acc_v, [jnp.full((L,), b, jnp.int32)], part)

    for k in range(NSLOT):
        issue(k, k)

    def round_body(r, carry):
        for k in range(NSLOT):
            b = r * NSLOT + k
            consume(b, k)
            issue(b + NSLOT, k)
        return carry

    lax.fori_loop(0, NROUND - 1, round_body, 0)
    for k in range(NSLOT):
        consume((NROUND - 1) * NSLOT + k, k)

    pltpu.sync_copy(acc_v, dot_hbm.at[pl.ds(base_b, BPW)])


def _bias_kernel(users_hbm, items_hbm, ub_hbm, ib_hbm, dot_hbm,
                 out_hbm, idx_u, idx_i, ub_v, ib_v, dot_v, sem):
    wid = lax.axis_index("s") * NC + lax.axis_index("c")

    pltpu.sync_copy(users_hbm.at[wid], idx_u)
    pltpu.sync_copy(items_hbm.at[wid], idx_i)
    pltpu.sync_copy(dot_hbm.at[pl.ds(wid * BPW, BPW)], dot_v)

    copies = []
    for j in range(NCHUNK):
        rows = pl.ds(j * CHUNK, CHUNK)
        copies.append(pltpu.make_async_copy(
            ub_hbm.at[idx_u.at[j]], ub_v.at[rows], sem))
        copies.append(pltpu.make_async_copy(
            ib_hbm.at[idx_i.at[j]], ib_v.at[rows], sem))
    for c in copies:
        c.start()
    for c in copies:
        c.wait()

    def body(g, carry):
        s = pl.ds(pl.multiple_of(g * L, L), L)
        acc = dot_v[s] + ub_v[s] + ib_v[s]
        y = 1.0 / (1.0 + jnp.exp(-acc))
        dot_v[s] = y * (Y_HI - Y_LO) + Y_LO
        return carry

    lax.fori_loop(0, NGROUP, body, 0)
    pltpu.sync_copy(dot_v, out_hbm.at[pl.ds(wid * BPW, BPW)])


@jax.jit
def kernel(users, items, user_emb, item_emb, user_bias, item_bias):
    users = users.astype(jnp.int32)
    items = items.astype(jnp.int32)
    users_3d = users.reshape(NW, NCHUNK, CHUNK)
    items_3d = items.reshape(NW, NCHUNK, CHUNK)
    ub = user_bias.reshape(-1)
    ib = item_bias.reshape(-1)
    uet = user_emb.T
    iet = item_emb.T

    mesh = plsc.VectorSubcoreMesh(core_axis_name="c", subcore_axis_name="s")

    dot_run = pl.kernel(
        _dot_kernel,
        out_type=jax.ShapeDtypeStruct((B,), jnp.float32),
        mesh=mesh,
        compiler_params=pltpu.CompilerParams(
            needs_layout_passes=False, use_tc_tiling_on_sc=True),
        scratch_types=(
            [pltpu.VMEM((BPW,), jnp.int32)] * 2
            + [pltpu.VMEM((BPW,), jnp.float32)]
            + [pltpu.VMEM((D, CHUNK), jnp.float32)] * (2 * NSLOT)
            + [pltpu.SemaphoreType.DMA] * (2 * NSLOT)
        ),
    )
    dots = dot_run(users, items, uet, iet)

    bias_run = pl.kernel(
        _bias_kernel,
        out_type=jax.ShapeDtypeStruct((B,), jnp.float32),
        mesh=mesh,
        compiler_params=pltpu.CompilerParams(
            needs_layout_passes=False, use_tc_tiling_on_sc=False),
        scratch_types=[
            pltpu.VMEM((NCHUNK, CHUNK), jnp.int32),
            pltpu.VMEM((NCHUNK, CHUNK), jnp.int32),
            pltpu.VMEM((BPW,), jnp.float32),
            pltpu.VMEM((BPW,), jnp.float32),
            pltpu.VMEM((BPW,), jnp.float32),
            pltpu.SemaphoreType.DMA,
        ],
    )
    return bias_run(users_3d, items_3d, ub, ib, dots)
